# Initial kernel scaffold; baseline (speedup 1.0000x reference)
import jax
import jax.numpy as jnp
from jax.experimental import pallas as pl


def kernel(features, rois):
    raise NotImplementedError("write your pallas kernel here")


# speedup vs baseline: 1.1786x; 1.1786x over previous
import functools

import jax
import jax.numpy as jnp
from jax import lax
from jax.experimental import pallas as pl
from jax.experimental.pallas import tpu as pltpu
from jax.experimental.pallas import tpu_sc as plsc

N, C, H, W = 4, 96, 224, 224
SPATIAL_SCALE_CONST = 1.0

NWORKERS = 32
K = 128
HB = 8


CP = 128


def _transpose_body(f_ref, o_ref):
    blk = f_ref[0]
    blk = blk.reshape(C, HB * W)
    blk = blk.T
    o_ref[...] = jnp.concatenate(
        [blk, jnp.zeros((HB * W, CP - C), jnp.float32)], axis=1)


def _build_table(features):
    grid = (N, H // HB)
    return pl.pallas_call(
        _transpose_body,
        grid=grid,
        in_specs=[pl.BlockSpec((1, C, HB, W), lambda n, h: (n, 0, h, 0))],
        out_specs=pl.BlockSpec((HB * W, CP), lambda n, h: (n * (H // HB) + h, 0)),
        out_shape=jax.ShapeDtypeStruct((N * H * W, CP), jnp.float32),
    )(features)


def _make_sc_interp(r_pad):
    pb = r_pad // NWORKERS
    n_chunks = pb // K
    mesh = plsc.VectorSubcoreMesh(core_axis_name="c", subcore_axis_name="s")

    @functools.partial(
        pl.kernel,
        mesh=mesh,
        out_type=jax.ShapeDtypeStruct((r_pad, C), jnp.float32),
        scratch_types=[
            pltpu.VMEM((K,), jnp.float32),
            pltpu.VMEM((K,), jnp.float32),
            pltpu.VMEM((K,), jnp.float32),
            pltpu.VMEM((K,), jnp.int32),
            pltpu.VMEM((K,), jnp.int32),
            pltpu.VMEM((K,), jnp.int32),
            pltpu.VMEM((K,), jnp.int32),
            pltpu.VMEM((K,), jnp.float32),
            pltpu.VMEM((K,), jnp.float32),
            pltpu.VMEM((K,), jnp.float32),
            pltpu.VMEM((K,), jnp.float32),
            pltpu.VMEM((K, CP), jnp.float32),
            pltpu.VMEM((K, CP), jnp.float32),
            pltpu.VMEM((K, CP), jnp.float32),
            pltpu.VMEM((K, CP), jnp.float32),
            pltpu.VMEM((K, C), jnp.float32),
            pltpu.SemaphoreType.DMA,
        ],
    )
    def sc_interp(xs_hbm, ys_hbm, bs_hbm, table_hbm, out_hbm,
                  xs_v, ys_v, bs_v,
                  i00, i01, i10, i11,
                  w00, w01, w10, w11,
                  r00, r01, r10, r11,
                  out_v, sem):
        wid = lax.axis_index("s") * 2 + lax.axis_index("c")

        def chunk(t, carry):
            base = wid * pb + t * K
            pltpu.sync_copy(xs_hbm.at[pl.ds(base, K)], xs_v)
            pltpu.sync_copy(ys_hbm.at[pl.ds(base, K)], ys_v)
            pltpu.sync_copy(bs_hbm.at[pl.ds(base, K)], bs_v)

            for j in range(K // 16):
                s = pl.ds(j * 16, 16)
                x = jnp.minimum(jnp.maximum(xs_v[s] * SPATIAL_SCALE_CONST, 0.0),
                                float(W - 1))
                y = jnp.minimum(jnp.maximum(ys_v[s] * SPATIAL_SCALE_CONST, 0.0),
                                float(H - 1))
                b = bs_v[s].astype(jnp.int32)
                x0 = x.astype(jnp.int32)
                y0 = y.astype(jnp.int32)
                lx = x - x0.astype(jnp.float32)
                ly = y - y0.astype(jnp.float32)
                dx = jnp.where(x0 < W - 1, 1, 0)
                dy = jnp.where(y0 < H - 1, W, 0)
                ib = (b * H + y0) * W + x0
                i00[s] = ib
                i01[s] = ib + dx
                i10[s] = ib + dy
                i11[s] = ib + dy + dx
                hx = 1.0 - lx
                hy = 1.0 - ly
                w00[s] = hy * hx
                w01[s] = hy * lx
                w10[s] = ly * hx
                w11[s] = ly * lx

            cp0 = pltpu.async_copy(table_hbm.at[i00], r00, sem)
            cp1 = pltpu.async_copy(table_hbm.at[i01], r01, sem)
            cp2 = pltpu.async_copy(table_hbm.at[i10], r10, sem)
            cp3 = pltpu.async_copy(table_hbm.at[i11], r11, sem)
            cp0.wait()
            cp1.wait()
            cp2.wait()
            cp3.wait()

            def point_group(q, carry2):
                qb = q * 16
                wv0 = w00[pl.ds(qb, 16)]
                wv1 = w01[pl.ds(qb, 16)]
                wv2 = w10[pl.ds(qb, 16)]
                wv3 = w11[pl.ds(qb, 16)]
                for lane in range(16):
                    p = qb + lane
                    a0 = wv0[lane]
                    a1 = wv1[lane]
                    a2 = wv2[lane]
                    a3 = wv3[lane]
                    for g in range(C // 16):
                        sg = pl.ds(g * 16, 16)
                        out_v[p, sg] = (a0 * r00[p, sg] + a1 * r01[p, sg]
                                        + a2 * r10[p, sg] + a3 * r11[p, sg])
                return carry2

            lax.fori_loop(0, K // 16, point_group, 0)
            pltpu.sync_copy(out_v, out_hbm.at[pl.ds(base, K)])
            return carry

        lax.fori_loop(0, n_chunks, chunk, 0)

    return sc_interp


def kernel(features, rois):
    r = rois.shape[0]
    chunk_stride = NWORKERS * K
    r_pad = ((r + chunk_stride - 1) // chunk_stride) * chunk_stride

    table = _build_table(features)

    bs = rois[:, 0]
    xs = rois[:, 1]
    ys = rois[:, 2]
    pad = r_pad - r
    if pad:
        bs = jnp.concatenate([bs, jnp.zeros((pad,), jnp.float32)])
        xs = jnp.concatenate([xs, jnp.zeros((pad,), jnp.float32)])
        ys = jnp.concatenate([ys, jnp.zeros((pad,), jnp.float32)])

    out = _make_sc_interp(r_pad)(xs, ys, bs, table)
    return out[:r]


# speedup vs baseline: 1.6106x; 1.3665x over previous
import functools

import jax
import jax.numpy as jnp
from jax import lax
from jax.experimental import pallas as pl
from jax.experimental.pallas import tpu as pltpu
from jax.experimental.pallas import tpu_sc as plsc

N, C, H, W = 4, 96, 224, 224
SPATIAL_SCALE_CONST = 1.0

NWORKERS = 32
K = 64
CP = 128
HB = 8


def _transpose_body(f_ref, o_ref):
    blk = f_ref[0]
    blk = blk.reshape(C, HB * W)
    blk = blk.T
    o_ref[...] = jnp.concatenate(
        [blk, jnp.zeros((HB * W, CP - C), jnp.float32)], axis=1)


def _build_table(features):
    grid = (N, H // HB)
    return pl.pallas_call(
        _transpose_body,
        grid=grid,
        in_specs=[pl.BlockSpec((1, C, HB, W), lambda n, h: (n, 0, h, 0))],
        out_specs=pl.BlockSpec((HB * W, CP), lambda n, h: (n * (H // HB) + h, 0)),
        out_shape=jax.ShapeDtypeStruct((N * H * W, CP), jnp.float32),
    )(features)


def _make_sc_interp(r, r_pad):
    pb = r_pad // NWORKERS
    nc = pb // K
    assert nc % 2 == 0 and nc >= 4
    mesh = plsc.VectorSubcoreMesh(core_axis_name="c", subcore_axis_name="s")

    scratch = [pltpu.VMEM((K,), jnp.float32) for _ in range(3)]
    scratch += [pltpu.VMEM((K,), jnp.int32) for _ in range(8)]
    scratch += [pltpu.VMEM((K,), jnp.float32) for _ in range(8)]
    scratch += [pltpu.VMEM((K, CP), jnp.float32) for _ in range(8)]
    scratch += [pltpu.VMEM((K, C), jnp.float32) for _ in range(2)]
    scratch += [pltpu.SemaphoreType.DMA for _ in range(4)]

    @functools.partial(
        pl.kernel,
        mesh=mesh,
        out_type=jax.ShapeDtypeStruct((r, C), jnp.float32),
        scratch_types=scratch,
    )
    def sc_interp(bs_hbm, xs_hbm, ys_hbm, table_hbm, out_hbm, bs_v, xs_v, ys_v, *rest):
        idx = [rest[0:4], rest[4:8]]
        wts = [rest[8:12], rest[12:16]]
        rows = [rest[16:20], rest[20:24]]
        outv = [rest[24], rest[25]]
        gsem = [rest[26], rest[27]]
        osem = [rest[28], rest[29]]

        wid = lax.axis_index("s") * 2 + lax.axis_index("c")
        wbase = wid * pb

        def stage(s, base):
            pltpu.sync_copy(bs_hbm.at[pl.ds(base, K)], bs_v)
            pltpu.sync_copy(xs_hbm.at[pl.ds(base, K)], xs_v)
            pltpu.sync_copy(ys_hbm.at[pl.ds(base, K)], ys_v)
            for j in range(K // 16):
                sl = pl.ds(j * 16, 16)
                b = bs_v[sl].astype(jnp.int32)
                x = jnp.minimum(jnp.maximum(xs_v[sl] * SPATIAL_SCALE_CONST,
                                            0.0), float(W - 1))
                y = jnp.minimum(jnp.maximum(ys_v[sl] * SPATIAL_SCALE_CONST,
                                            0.0), float(H - 1))
                x0 = x.astype(jnp.int32)
                y0 = y.astype(jnp.int32)
                lx = x - x0.astype(jnp.float32)
                ly = y - y0.astype(jnp.float32)
                dx = jnp.where(x0 < W - 1, 1, 0)
                dy = jnp.where(y0 < H - 1, W, 0)
                ib = (b * H + y0) * W + x0
                idx[s][0][sl] = ib
                idx[s][1][sl] = ib + dx
                idx[s][2][sl] = ib + dy
                idx[s][3][sl] = ib + dy + dx
                hx = 1.0 - lx
                hy = 1.0 - ly
                wts[s][0][sl] = hy * hx
                wts[s][1][sl] = hy * lx
                wts[s][2][sl] = ly * hx
                wts[s][3][sl] = ly * lx
            for c in range(4):
                pltpu.async_copy(table_hbm.at[idx[s][c]], rows[s][c], gsem[s])

        def process(s, base, t):
            for c in range(4):
                pltpu.make_async_copy(
                    table_hbm.at[idx[s][c]], rows[s][c], gsem[s]).wait()

            prev_valid = (t >= 2) & (base - 2 * K < r)

            @pl.when(prev_valid)
            def _():
                pltpu.make_async_copy(
                    outv[s], out_hbm.at[pl.ds(0, K)], osem[s]).wait()

            r0, r1, r2, r3 = rows[s]
            w0, w1, w2, w3 = wts[s]
            ov = outv[s]

            def point_group(q, carry2):
                qb = q * 16
                wv0 = w0[pl.ds(qb, 16)]
                wv1 = w1[pl.ds(qb, 16)]
                wv2 = w2[pl.ds(qb, 16)]
                wv3 = w3[pl.ds(qb, 16)]
                for lane in range(16):
                    p = qb + lane
                    a0 = wv0[lane]
                    a1 = wv1[lane]
                    a2 = wv2[lane]
                    a3 = wv3[lane]
                    for g in range(C // 16):
                        sg = pl.ds(g * 16, 16)
                        ov[p, sg] = (a0 * r0[p, sg] + a1 * r1[p, sg]
                                     + a2 * r2[p, sg] + a3 * r3[p, sg])
                return carry2

            lax.fori_loop(0, K // 16, point_group, 0)

            @pl.when(base < r)
            def _():
                pltpu.async_copy(ov, out_hbm.at[pl.ds(base, K)], osem[s])

        stage(0, wbase)
        stage(1, wbase + K)

        def pair(p2, carry):
            t0 = 2 * p2
            b0 = wbase + t0 * K
            process(0, b0, t0)

            @pl.when(t0 + 2 < nc)
            def _():
                stage(0, b0 + 2 * K)

            t1 = t0 + 1
            b1 = b0 + K
            process(1, b1, t1)

            @pl.when(t1 + 2 < nc)
            def _():
                stage(1, b1 + 2 * K)

            return carry

        lax.fori_loop(0, nc // 2, pair, 0)

        for s in range(2):
            @pl.when(wbase + (nc - 2 + s) * K < r)
            def _():
                pltpu.make_async_copy(
                    outv[s], out_hbm.at[pl.ds(0, K)], osem[s]).wait()

    return sc_interp


def kernel(features, rois):
    r = rois.shape[0]
    chunk_stride = NWORKERS * K * 2
    r_pad = ((r + chunk_stride - 1) // chunk_stride) * chunk_stride

    table = _build_table(features)

    bs = rois[:, 0]
    xs = rois[:, 1]
    ys = rois[:, 2]
    pad = r_pad - r
    if pad:
        z = jnp.zeros((pad,), jnp.float32)
        bs = jnp.concatenate([bs, z])
        xs = jnp.concatenate([xs, z])
        ys = jnp.concatenate([ys, z])

    return _make_sc_interp(r, r_pad)(bs, xs, ys, table)


# speedup vs baseline: 1.9134x; 1.1880x over previous
import functools

import jax
import jax.numpy as jnp
from jax import lax
from jax.experimental import pallas as pl
from jax.experimental.pallas import tpu as pltpu
from jax.experimental.pallas import tpu_sc as plsc

N, C, H, W = 4, 96, 224, 224
SPATIAL_SCALE_CONST = 1.0

NWORKERS = 32
K = 64
CP = 128
HB = 8


def _transpose_body(f_ref, o_ref):
    blk = f_ref[0]
    blk = blk.reshape(C, HB * W)
    blk = blk.T
    o_ref[...] = jnp.concatenate(
        [blk, jnp.zeros((HB * W, CP - C), jnp.float32)], axis=1)


def _build_table(features):
    grid = (N, H // HB)
    return pl.pallas_call(
        _transpose_body,
        grid=grid,
        in_specs=[pl.BlockSpec((1, C, HB, W), lambda n, h: (n, 0, h, 0))],
        out_specs=pl.BlockSpec((HB * W, CP), lambda n, h: (n * (H // HB) + h, 0)),
        out_shape=jax.ShapeDtypeStruct((N * H * W, CP), jnp.float32),
    )(features)


def _make_sc_interp(r, r_pad):
    pb = r_pad // NWORKERS
    nc = pb // K
    assert nc % 2 == 0 and nc >= 4
    mesh = plsc.VectorSubcoreMesh(core_axis_name="c", subcore_axis_name="s")

    scratch = [pltpu.VMEM((pb,), jnp.float32) for _ in range(3)]
    scratch += [pltpu.VMEM((2 * K,), jnp.int32) for _ in range(4)]
    scratch += [pltpu.VMEM((K,), jnp.float32) for _ in range(8)]
    scratch += [pltpu.VMEM((2 * K, CP), jnp.float32) for _ in range(4)]
    scratch += [pltpu.VMEM((K, C), jnp.float32) for _ in range(2)]
    scratch += [pltpu.SemaphoreType.DMA for _ in range(4)]

    @functools.partial(
        pl.kernel,
        mesh=mesh,
        out_type=jax.ShapeDtypeStruct((r, C), jnp.float32),
        scratch_types=scratch,
    )
    def sc_interp(bs_hbm, xs_hbm, ys_hbm, table_hbm, out_hbm,
                  bs_v, xs_v, ys_v, *rest):
        idx = [rest[0:2], rest[2:4]]
        wts = [rest[4:8], rest[8:12]]
        rows = [rest[12:14], rest[14:16]]
        outv = [rest[16], rest[17]]
        gsem = [rest[18], rest[19]]
        osem = [rest[20], rest[21]]

        wid = lax.axis_index("s") * 2 + lax.axis_index("c")
        wbase = wid * pb

        pltpu.sync_copy(bs_hbm.at[pl.ds(wbase, pb)], bs_v)
        pltpu.sync_copy(xs_hbm.at[pl.ds(wbase, pb)], xs_v)
        pltpu.sync_copy(ys_hbm.at[pl.ds(wbase, pb)], ys_v)

        def stage(s, off):
            for j in range(K // 16):
                sl = pl.ds(off + j * 16, 16)
                b = bs_v[sl].astype(jnp.int32)
                x = jnp.minimum(jnp.maximum(xs_v[sl] * SPATIAL_SCALE_CONST,
                                            0.0), float(W - 1))
                y = jnp.minimum(jnp.maximum(ys_v[sl] * SPATIAL_SCALE_CONST,
                                            0.0), float(H - 1))
                x0 = x.astype(jnp.int32)
                y0 = y.astype(jnp.int32)
                lx = x - x0.astype(jnp.float32)
                ly = y - y0.astype(jnp.float32)
                dx = jnp.where(x0 < W - 1, 1, 0)
                dy = jnp.where(y0 < H - 1, W, 0)
                ib = (b * H + y0) * W + x0
                sj = pl.ds(j * 16, 16)
                sj2 = pl.ds(K + j * 16, 16)
                idx[s][0][sj] = ib
                idx[s][0][sj2] = ib + dx
                idx[s][1][sj] = ib + dy
                idx[s][1][sj2] = ib + dy + dx
                hx = 1.0 - lx
                hy = 1.0 - ly
                wts[s][0][sj] = hy * hx
                wts[s][1][sj] = hy * lx
                wts[s][2][sj] = ly * hx
                wts[s][3][sj] = ly * lx
            for c in range(2):
                pltpu.async_copy(table_hbm.at[idx[s][c]], rows[s][c], gsem[s])

        def process(s, base, t):
            for c in range(2):
                pltpu.make_async_copy(
                    table_hbm.at[idx[s][c]], rows[s][c], gsem[s]).wait()

            prev_valid = (t >= 2) & (base - 2 * K < r)

            @pl.when(prev_valid)
            def _():
                pltpu.make_async_copy(
                    outv[s], out_hbm.at[pl.ds(0, K)], osem[s]).wait()

            ra, rb = rows[s]
            w0, w1, w2, w3 = wts[s]
            ov = outv[s]

            def point_group(q, carry2):
                qb = q * 16
                wv0 = w0[pl.ds(qb, 16)]
                wv1 = w1[pl.ds(qb, 16)]
                wv2 = w2[pl.ds(qb, 16)]
                wv3 = w3[pl.ds(qb, 16)]
                for lane in range(16):
                    p = qb + lane
                    a0 = wv0[lane]
                    a1 = wv1[lane]
                    a2 = wv2[lane]
                    a3 = wv3[lane]
                    for g in range(C // 16):
                        sg = pl.ds(g * 16, 16)
                        ov[p, sg] = (a0 * ra[p, sg] + a1 * ra[K + p, sg]
                                     + a2 * rb[p, sg] + a3 * rb[K + p, sg])
                return carry2

            lax.fori_loop(0, K // 16, point_group, 0)

            @pl.when(base < r)
            def _():
                pltpu.async_copy(ov, out_hbm.at[pl.ds(base, K)], osem[s])

        stage(0, 0)
        stage(1, K)

        def pair(p2, carry):
            t0 = 2 * p2
            off0 = t0 * K
            b0 = wbase + off0
            process(0, b0, t0)

            @pl.when(t0 + 2 < nc)
            def _():
                stage(0, off0 + 2 * K)

            t1 = t0 + 1
            b1 = b0 + K
            process(1, b1, t1)

            @pl.when(t1 + 2 < nc)
            def _():
                stage(1, off0 + 3 * K)

            return carry

        lax.fori_loop(0, nc // 2, pair, 0)

        for s in range(2):
            @pl.when(wbase + (nc - 2 + s) * K < r)
            def _():
                pltpu.make_async_copy(
                    outv[s], out_hbm.at[pl.ds(0, K)], osem[s]).wait()

    return sc_interp


def kernel(features, rois):
    r = rois.shape[0]
    chunk_stride = NWORKERS * K * 2
    r_pad = ((r + chunk_stride - 1) // chunk_stride) * chunk_stride

    table = _build_table(features)

    bs = rois[:, 0]
    xs = rois[:, 1]
    ys = rois[:, 2]
    pad = r_pad - r
    if pad:
        z = jnp.zeros((pad,), jnp.float32)
        bs = jnp.concatenate([bs, z])
        xs = jnp.concatenate([xs, z])
        ys = jnp.concatenate([ys, z])

    return _make_sc_interp(r, r_pad)(bs, xs, ys, table)
